# baseline (speedup 1.0000x reference)
import functools

import jax
import jax.numpy as jnp
from jax import lax
from jax.experimental import pallas as pl
from jax.experimental.pallas import tpu as pltpu
from jax.experimental.pallas import tpu_sc as plsc

_ENT_NUM = 1000000
_REL_NUM = 1000000
_DIM = 64
_BATCH = 16384
_MARGIN = 4.0
_ALPHA = 0.01

_NW = 32
_BPW = _BATCH // _NW
_CHUNK = 128
_NCHUNK = _BPW // _CHUNK
_NGRP = _CHUNK // 16


def _sc_body(idx_all, ent_emb, rel_emb, norm_vec, out,
             hp_b, rp_b, tp_b, vp_b, hn_b, rn_b, tn_b, vn_b,
             idx_v, out_v, sem):
  wid = lax.axis_index("s") * 2 + lax.axis_index("c")
  pltpu.sync_copy(idx_all.at[wid], idx_v)

  zero = jnp.zeros((16,), jnp.float32)
  hinge_acc = zero
  habs_acc = zero
  tabs_acc = zero
  rsq_acc = zero

  for c in range(_NCHUNK):
    cps = [
        pltpu.async_copy(ent_emb.at[idx_v.at[0, c]], hp_b, sem),
        pltpu.async_copy(rel_emb.at[idx_v.at[1, c]], rp_b, sem),
        pltpu.async_copy(ent_emb.at[idx_v.at[2, c]], tp_b, sem),
        pltpu.async_copy(norm_vec.at[idx_v.at[1, c]], vp_b, sem),
        pltpu.async_copy(ent_emb.at[idx_v.at[3, c]], hn_b, sem),
        pltpu.async_copy(rel_emb.at[idx_v.at[4, c]], rn_b, sem),
        pltpu.async_copy(ent_emb.at[idx_v.at[5, c]], tn_b, sem),
        pltpu.async_copy(norm_vec.at[idx_v.at[4, c]], vn_b, sem),
    ]
    for cp in cps:
      cp.wait()

    def group_body(g, carry):
      hinge_a, habs_a, tabs_a, rsq_a = carry
      rid = (g * 16 + jnp.arange(16, dtype=jnp.int32)).astype(jnp.int32)

      def pass1(j, cr):
        cp_a, cn_a, ha, ta = cr
        cj = jnp.full((16,), j, jnp.int32)
        hpv = plsc.load_gather(hp_b, [rid, cj])
        tpv = plsc.load_gather(tp_b, [rid, cj])
        vpv = plsc.load_gather(vp_b, [rid, cj])
        hnv = plsc.load_gather(hn_b, [rid, cj])
        tnv = plsc.load_gather(tn_b, [rid, cj])
        vnv = plsc.load_gather(vn_b, [rid, cj])
        cp_a = cp_a + (hpv - tpv) * vpv
        cn_a = cn_a + (hnv - tnv) * vnv
        ha = ha + jnp.abs(hpv) + jnp.abs(hnv)
        ta = ta + jnp.abs(tpv) + jnp.abs(tnv)
        return cp_a, cn_a, ha, ta

      cp_a, cn_a, habs_a, tabs_a = lax.fori_loop(
          0, _DIM, pass1, (zero, zero, habs_a, tabs_a))

      def pass2(j, cr):
        dp_a, dn_a, ra = cr
        cj = jnp.full((16,), j, jnp.int32)
        hpv = plsc.load_gather(hp_b, [rid, cj])
        tpv = plsc.load_gather(tp_b, [rid, cj])
        vpv = plsc.load_gather(vp_b, [rid, cj])
        rpv = plsc.load_gather(rp_b, [rid, cj])
        hnv = plsc.load_gather(hn_b, [rid, cj])
        tnv = plsc.load_gather(tn_b, [rid, cj])
        vnv = plsc.load_gather(vn_b, [rid, cj])
        rnv = plsc.load_gather(rn_b, [rid, cj])
        sp = hpv - tpv + rpv - cp_a * vpv
        sn = hnv - tnv + rnv - cn_a * vnv
        dp_a = dp_a + jnp.abs(sp)
        dn_a = dn_a + jnp.abs(sn)
        ra = ra + rpv * rpv + rnv * rnv
        return dp_a, dn_a, ra

      dp_a, dn_a, rsq_a = lax.fori_loop(
          0, _DIM, pass2, (zero, zero, rsq_a))

      hinge_a = hinge_a + jnp.maximum(0.0, dp_a - dn_a + _MARGIN)
      return hinge_a, habs_a, tabs_a, rsq_a

    hinge_acc, habs_acc, tabs_acc, rsq_acc = lax.fori_loop(
        0, _NGRP, group_body, (hinge_acc, habs_acc, tabs_acc, rsq_acc))

  out_v[0, :] = hinge_acc
  out_v[1, :] = habs_acc
  out_v[2, :] = tabs_acc
  out_v[3, :] = rsq_acc
  pltpu.sync_copy(out_v, out.at[wid])


@jax.jit
def kernel(pos_triplets, neg_triplets, ent_emb, rel_emb, norm_vec):
  pos = pos_triplets.astype(jnp.int32)
  neg = neg_triplets.astype(jnp.int32)
  cols = jnp.concatenate([pos.T, neg.T], axis=0)
  idx_all = cols.reshape(6, _NW, _NCHUNK, _CHUNK).transpose(1, 0, 2, 3)

  call = pl.kernel(
      _sc_body,
      out_type=jax.ShapeDtypeStruct((_NW, 4, 16), jnp.float32),
      mesh=plsc.VectorSubcoreMesh(core_axis_name="c", subcore_axis_name="s"),
      scratch_types=[
          pltpu.VMEM((_CHUNK, _DIM), jnp.float32),
          pltpu.VMEM((_CHUNK, _DIM), jnp.float32),
          pltpu.VMEM((_CHUNK, _DIM), jnp.float32),
          pltpu.VMEM((_CHUNK, _DIM), jnp.float32),
          pltpu.VMEM((_CHUNK, _DIM), jnp.float32),
          pltpu.VMEM((_CHUNK, _DIM), jnp.float32),
          pltpu.VMEM((_CHUNK, _DIM), jnp.float32),
          pltpu.VMEM((_CHUNK, _DIM), jnp.float32),
          pltpu.VMEM((6, _NCHUNK, _CHUNK), jnp.int32),
          pltpu.VMEM((4, 16), jnp.float32),
          pltpu.SemaphoreType.DMA,
      ],
      compiler_params=pltpu.CompilerParams(
          needs_layout_passes=False, use_tc_tiling_on_sc=False),
  )
  parts = call(idx_all, ent_emb, rel_emb, norm_vec)
  s = jnp.sum(parts, axis=(0, 2))
  loss = (s[0] / _BATCH
          + (_ALPHA / 3.0) * (s[1] / _BATCH + s[2] / _BATCH
                              + s[3] / (_BATCH * _DIM) - 4.0))
  return loss
